# Initial kernel scaffold; baseline (speedup 1.0000x reference)
import functools

import jax
import jax.numpy as jnp
from jax import lax
from jax.experimental import pallas as pl
from jax.experimental.pallas import tpu as pltpu
from jax.experimental.pallas import tpu_sc as plsc

VOCAB = 1000
BATCH = 4096
SEQ = 50

_info = plsc.get_sparse_core_info()
NC = _info.num_cores
NS = _info.num_subcores
NW = NC * NS
B_TOTAL = BATCH * SEQ
B_PER_W = B_TOTAL // NW
CHUNK = 50
N_CHUNKS = B_PER_W // CHUNK


def _emb_body(table_hbm, idx_hbm, out_hbm, idx_v, rows_v, gsem):
    c = lax.axis_index("c")
    s = lax.axis_index("s")
    wid = s * NC + c
    pltpu.sync_copy(idx_hbm.at[wid], idx_v)
    base = wid * B_PER_W

    def chunk(i, carry):
        pltpu.async_copy(table_hbm.at[idx_v.at[i]], rows_v, gsem).wait()
        pltpu.sync_copy(rows_v, out_hbm.at[pl.ds(base + i * CHUNK, CHUNK)])
        return carry

    lax.fori_loop(0, N_CHUNKS, chunk, 0)


@jax.jit
def _embed(idx, table):
    mesh = plsc.VectorSubcoreMesh(core_axis_name="c", subcore_axis_name="s")
    fn = pl.kernel(
        _emb_body,
        out_type=jax.ShapeDtypeStruct((B_TOTAL, VOCAB), jnp.float32),
        mesh=mesh,
        scratch_types=[
            pltpu.VMEM((N_CHUNKS, CHUNK), jnp.int32),
            pltpu.VMEM((CHUNK, VOCAB), jnp.float32),
            pltpu.SemaphoreType.DMA,
        ],
    )
    return fn(table, idx)


def kernel(inputs, targets, table):
    idx = inputs.astype(jnp.int32).reshape(NW, N_CHUNKS, CHUNK)
    out = _embed(idx, table)
    return out.reshape(BATCH, SEQ, VOCAB)


# speedup vs baseline: 1.0197x; 1.0197x over previous
import functools

import jax
import jax.numpy as jnp
from jax import lax
from jax.experimental import pallas as pl
from jax.experimental.pallas import tpu as pltpu
from jax.experimental.pallas import tpu_sc as plsc

VOCAB = 1000
BATCH = 4096
SEQ = 50

_info = plsc.get_sparse_core_info()
NC = _info.num_cores
NS = _info.num_subcores
NW = NC * NS
B_TOTAL = BATCH * SEQ
B_PER_W = B_TOTAL // NW
CHUNK = 64
N_CHUNKS = B_PER_W // CHUNK


def _emb_body(table_hbm, idx_hbm, out_hbm, idx_v, rows_v, gsem):
    c = lax.axis_index("c")
    s = lax.axis_index("s")
    wid = s * NC + c
    pltpu.sync_copy(idx_hbm.at[wid], idx_v)
    base = wid * B_PER_W

    def chunk(i, carry):
        pltpu.async_copy(table_hbm.at[idx_v.at[i]], rows_v, gsem).wait()
        pltpu.sync_copy(rows_v, out_hbm.at[pl.ds(base + i * CHUNK, CHUNK)])
        return carry

    lax.fori_loop(0, N_CHUNKS, chunk, 0)


@jax.jit
def _embed(idx, table):
    mesh = plsc.VectorSubcoreMesh(core_axis_name="c", subcore_axis_name="s")
    fn = pl.kernel(
        _emb_body,
        out_type=jax.ShapeDtypeStruct((B_TOTAL, VOCAB), jnp.float32),
        mesh=mesh,
        scratch_types=[
            pltpu.VMEM((N_CHUNKS, CHUNK), jnp.int32),
            pltpu.VMEM((CHUNK, VOCAB), jnp.float32),
            pltpu.SemaphoreType.DMA,
        ],
        compiler_params=pltpu.CompilerParams(use_tc_tiling_on_sc=False),
    )
    return fn(table, idx)


def kernel(inputs, targets, table):
    idx = inputs.astype(jnp.int32).reshape(NW, N_CHUNKS, CHUNK)
    out = _embed(idx, table)
    return out.reshape(BATCH, SEQ, VOCAB)


# speedup vs baseline: 1.0343x; 1.0143x over previous
import functools

import jax
import jax.numpy as jnp
from jax import lax
from jax.experimental import pallas as pl
from jax.experimental.pallas import tpu as pltpu
from jax.experimental.pallas import tpu_sc as plsc

VOCAB = 1000
BATCH = 4096
SEQ = 50

_info = plsc.get_sparse_core_info()
NC = _info.num_cores
NS = _info.num_subcores
NW = NC * NS
B_TOTAL = BATCH * SEQ
B_PER_W = B_TOTAL // NW
CHUNK = 40
N_CHUNKS = B_PER_W // CHUNK


def _emb_body(table_hbm, idx_hbm, out_hbm,
              idx_v, rows0, rows1, g0, g1, s0, s1):
    c = lax.axis_index("c")
    s = lax.axis_index("s")
    wid = s * NC + c
    pltpu.sync_copy(idx_hbm.at[wid], idx_v)
    base = wid * B_PER_W

    bufs = (rows0, rows1)
    gsems = (g0, g1)
    ssems = (s0, s1)

    def gather_cp(i, b):
        return pltpu.make_async_copy(table_hbm.at[idx_v.at[i]], bufs[b],
                                     gsems[b])

    def scatter_cp(i, b):
        return pltpu.make_async_copy(
            bufs[b], out_hbm.at[pl.ds(base + i * CHUNK, CHUNK)], ssems[b])

    gather_cp(0, 0).start()

    def pair(j, carry):
        for b in range(2):
            i = 2 * j + b
            other = 1 - b
            gather_cp(i, b).wait()
            scatter_cp(i, b).start()

            @pl.when(i + 1 < N_CHUNKS)
            def _():
                @pl.when(i >= 1)
                def _():
                    scatter_cp(i - 1, other).wait()
                gather_cp(i + 1, other).start()
        return carry

    lax.fori_loop(0, N_CHUNKS // 2, pair, 0)
    scatter_cp(N_CHUNKS - 2, (N_CHUNKS - 2) % 2).wait()
    scatter_cp(N_CHUNKS - 1, (N_CHUNKS - 1) % 2).wait()


@jax.jit
def _embed(idx, table):
    mesh = plsc.VectorSubcoreMesh(core_axis_name="c", subcore_axis_name="s")
    fn = pl.kernel(
        _emb_body,
        out_type=jax.ShapeDtypeStruct((B_TOTAL, VOCAB), jnp.float32),
        mesh=mesh,
        scratch_types=[
            pltpu.VMEM((N_CHUNKS, CHUNK), jnp.int32),
            pltpu.VMEM((CHUNK, VOCAB), jnp.float32),
            pltpu.VMEM((CHUNK, VOCAB), jnp.float32),
            pltpu.SemaphoreType.DMA,
            pltpu.SemaphoreType.DMA,
            pltpu.SemaphoreType.DMA,
            pltpu.SemaphoreType.DMA,
        ],
        compiler_params=pltpu.CompilerParams(use_tc_tiling_on_sc=False),
    )
    return fn(table, idx)


def kernel(inputs, targets, table):
    idx = inputs.astype(jnp.int32).reshape(NW, N_CHUNKS, CHUNK)
    out = _embed(idx, table)
    return out.reshape(BATCH, SEQ, VOCAB)


# speedup vs baseline: 1.1661x; 1.1274x over previous
import functools

import jax
import jax.numpy as jnp
from jax import lax
from jax.experimental import pallas as pl
from jax.experimental.pallas import tpu as pltpu
from jax.experimental.pallas import tpu_sc as plsc

VOCAB = 1000
BATCH = 4096
SEQ = 50

_info = plsc.get_sparse_core_info()
NC = _info.num_cores
NS = _info.num_subcores
NW = NC * NS
B_TOTAL = BATCH * SEQ
B_PER_W = B_TOTAL // NW
CHUNK = 16
N_CHUNKS = B_PER_W // CHUNK
NBUF = 3
LAG = 2
STAGE = 64


def _emb_body(table_hbm, idx_hbm, out_hbm,
              tab_sp, idx_v, rows0, rows1, rows2, g0, g1, g2, s0, s1, s2):
    c = lax.axis_index("c")
    s = lax.axis_index("s")
    wid = s * NC + c

    @pl.when(s < NS - 1)
    def _():
        pltpu.sync_copy(table_hbm.at[pl.ds(s * STAGE, STAGE)],
                        tab_sp.at[pl.ds(s * STAGE, STAGE)])

    @pl.when(s == NS - 1)
    def _():
        pltpu.sync_copy(table_hbm.at[pl.ds((NS - 1) * STAGE,
                                           VOCAB - (NS - 1) * STAGE)],
                        tab_sp.at[pl.ds((NS - 1) * STAGE,
                                        VOCAB - (NS - 1) * STAGE)])

    pltpu.sync_copy(idx_hbm.at[wid], idx_v)
    plsc.subcore_barrier()

    base = wid * B_PER_W
    bufs = (rows0, rows1, rows2)
    gsems = (g0, g1, g2)
    ssems = (s0, s1, s2)

    def gather_cp(i, b):
        return pltpu.make_async_copy(tab_sp.at[idx_v.at[i]], bufs[b],
                                     gsems[b])

    def scatter_cp(i, b):
        return pltpu.make_async_copy(
            bufs[b], out_hbm.at[pl.ds(base + i * CHUNK, CHUNK)], ssems[b])

    def step(jj, carry):
        for u in range(NBUF):
            i = jj * NBUF + u
            b = u

            @pl.when(i < N_CHUNKS)
            def _():
                @pl.when(i >= NBUF)
                def _():
                    scatter_cp(i - NBUF, b).wait()
                gather_cp(i, b).start()

            j = i - LAG
            bs = (u - LAG) % NBUF

            @pl.when(jnp.logical_and(j >= 0, j < N_CHUNKS))
            def _():
                gather_cp(j, bs).wait()
                scatter_cp(j, bs).start()
        return carry

    total = N_CHUNKS + LAG
    lax.fori_loop(0, (total + NBUF - 1) // NBUF, step, 0)

    for k in range(NBUF):
        i = N_CHUNKS - NBUF + k
        scatter_cp(i, i % NBUF).wait()


@jax.jit
def _embed(idx, table):
    mesh = plsc.VectorSubcoreMesh(core_axis_name="c", subcore_axis_name="s")
    fn = pl.kernel(
        _emb_body,
        out_type=jax.ShapeDtypeStruct((B_TOTAL, VOCAB), jnp.float32),
        mesh=mesh,
        scratch_types=[
            pltpu.VMEM_SHARED((VOCAB, VOCAB), jnp.float32),
            pltpu.VMEM((N_CHUNKS, CHUNK), jnp.int32),
            pltpu.VMEM((CHUNK, VOCAB), jnp.float32),
            pltpu.VMEM((CHUNK, VOCAB), jnp.float32),
            pltpu.VMEM((CHUNK, VOCAB), jnp.float32),
            pltpu.SemaphoreType.DMA,
            pltpu.SemaphoreType.DMA,
            pltpu.SemaphoreType.DMA,
            pltpu.SemaphoreType.DMA,
            pltpu.SemaphoreType.DMA,
            pltpu.SemaphoreType.DMA,
        ],
        compiler_params=pltpu.CompilerParams(use_tc_tiling_on_sc=False),
    )
    return fn(table, idx)


def kernel(inputs, targets, table):
    idx = inputs.astype(jnp.int32).reshape(NW, N_CHUNKS, CHUNK)
    out = _embed(idx, table)
    return out.reshape(BATCH, SEQ, VOCAB)


# speedup vs baseline: 1.2631x; 1.0832x over previous
import functools

import jax
import jax.numpy as jnp
from jax import lax
from jax.experimental import pallas as pl
from jax.experimental.pallas import tpu as pltpu
from jax.experimental.pallas import tpu_sc as plsc

VOCAB = 1000
BATCH = 4096
SEQ = 50

_info = plsc.get_sparse_core_info()
NC = _info.num_cores
NS = _info.num_subcores
NW = NC * NS
BG = BATCH // NW
W = 40
K = VOCAB // W
NVG = W // 8
N_CHUNKS = SEQ * K
NSEG = VOCAB * K
SROWS = 1568


def _emb_body(tab_hbm, idx_hbm, out_hbm,
              tab_sp, idx_v, si0, si1, segs0, segs1, xb0, xb1,
              g0, g1, w0, w1):
    c = lax.axis_index("c")
    s = lax.axis_index("s")
    wid = s * NC + c

    @pl.when(s < NS - 1)
    def _():
        pltpu.sync_copy(tab_hbm.at[pl.ds(s * SROWS, SROWS)],
                        tab_sp.at[pl.ds(s * SROWS, SROWS)])

    @pl.when(s == NS - 1)
    def _():
        pltpu.sync_copy(tab_hbm.at[pl.ds((NS - 1) * SROWS,
                                         NSEG - (NS - 1) * SROWS)],
                        tab_sp.at[pl.ds((NS - 1) * SROWS,
                                        NSEG - (NS - 1) * SROWS)])

    pltpu.sync_copy(idx_hbm.at[:, wid], idx_v)
    plsc.subcore_barrier()

    sis = (si0, si1)
    segss = (segs0, segs1)
    xbs = (xb0, xb1)
    gsems = (g0, g1)
    wsems = (w0, w1)

    lane = lax.iota(jnp.int32, 16)
    row_idx = tuple(lane + (g * 16) for g in range(8))

    def fill_seg_idx(i, b):
        ss = i // K
        kk = i % K
        for g in range(8):
            r = idx_v[ss, pl.ds(g * 16, 16)]
            sis[b][pl.ds(g * 16, 16)] = r * K + kk

    def gather_cp(b):
        return pltpu.make_async_copy(tab_sp.at[sis[b]], segss[b], gsems[b])

    def write_cp(i, b):
        ss = i // K
        kk = i % K
        return pltpu.make_async_copy(
            xbs[b], out_hbm.at[ss, pl.ds(kk * NVG, NVG), wid], wsems[b])

    def transpose(b):
        def cbody(col, carry):
            for g in range(8):
                v = plsc.load_gather(
                    segss[b], [row_idx[g], jnp.full((16,), col, jnp.int32)])
                xbs[b][col // 8, col % 8, pl.ds(g * 16, 16)] = v
            return carry
        lax.fori_loop(0, W, cbody, 0)

    fill_seg_idx(0, 0)
    gather_cp(0).start()

    def step(jj, carry):
        for u in range(2):
            i = jj * 2 + u
            b = u
            other = 1 - u

            @pl.when(i + 1 < N_CHUNKS)
            def _():
                fill_seg_idx(i + 1, other)
                gather_cp(other).start()

            @pl.when(i >= 2)
            def _():
                write_cp(i - 2, b).wait()
            gather_cp(b).wait()
            transpose(b)
            write_cp(i, b).start()
        return carry

    lax.fori_loop(0, N_CHUNKS // 2, step, 0)
    write_cp(N_CHUNKS - 2, 0).wait()
    write_cp(N_CHUNKS - 1, 1).wait()


@jax.jit
def _embed(idx_t, tab_view):
    mesh = plsc.VectorSubcoreMesh(core_axis_name="c", subcore_axis_name="s")
    fn = pl.kernel(
        _emb_body,
        out_type=jax.ShapeDtypeStruct((SEQ, VOCAB // 8, NW, 8, BG),
                                      jnp.float32),
        mesh=mesh,
        scratch_types=[
            pltpu.VMEM_SHARED((NSEG, W), jnp.float32),
            pltpu.VMEM((SEQ, BG), jnp.int32),
            pltpu.VMEM((BG,), jnp.int32),
            pltpu.VMEM((BG,), jnp.int32),
            pltpu.VMEM((BG, W), jnp.float32),
            pltpu.VMEM((BG, W), jnp.float32),
            pltpu.VMEM((NVG, 8, BG), jnp.float32),
            pltpu.VMEM((NVG, 8, BG), jnp.float32),
            pltpu.SemaphoreType.DMA,
            pltpu.SemaphoreType.DMA,
            pltpu.SemaphoreType.DMA,
            pltpu.SemaphoreType.DMA,
        ],
        compiler_params=pltpu.CompilerParams(use_tc_tiling_on_sc=False,
                                             needs_layout_passes=False),
    )
    return fn(tab_view, idx_t)


def kernel(inputs, targets, table):
    idx_t = inputs.astype(jnp.int32).T.reshape(SEQ, NW, BG)
    tab_view = table.reshape(NSEG, W)
    x = _embed(idx_t, tab_view)
    return x.transpose(2, 4, 0, 1, 3).reshape(BATCH, SEQ, VOCAB)


# speedup vs baseline: 1.2671x; 1.0032x over previous
import functools

import jax
import jax.numpy as jnp
from jax import lax
from jax.experimental import pallas as pl
from jax.experimental.pallas import tpu as pltpu
from jax.experimental.pallas import tpu_sc as plsc

VOCAB = 1000
BATCH = 4096
SEQ = 50

_info = plsc.get_sparse_core_info()
NC = _info.num_cores
NS = _info.num_subcores
NW = NC * NS
BG = BATCH // NW
W = 200
K = VOCAB // W
NVG = W // 8
N_CHUNKS = SEQ * K
NSEG = VOCAB * K


def _emb_body(tab_hbm, idx_hbm, out_hbm,
              idx_v, si0, si1, segs0, segs1, xb0, xb1,
              g0, g1, w0, w1):
    c = lax.axis_index("c")
    s = lax.axis_index("s")
    wid = s * NC + c

    pltpu.sync_copy(idx_hbm.at[:, wid], idx_v)

    sis = (si0, si1)
    segss = (segs0, segs1)
    xbs = (xb0, xb1)
    gsems = (g0, g1)
    wsems = (w0, w1)

    lane = lax.iota(jnp.int32, 16)
    row_idx = tuple(lane + (g * 16) for g in range(8))

    def fill_seg_idx(i, b):
        ss = i // K
        kk = i % K
        for g in range(8):
            r = idx_v[ss, pl.ds(g * 16, 16)]
            sis[b][pl.ds(g * 16, 16)] = r * K + kk

    def gather_cp(b):
        return pltpu.make_async_copy(tab_hbm.at[sis[b]], segss[b], gsems[b])

    def write_cp(i, b):
        ss = i // K
        kk = i % K
        return pltpu.make_async_copy(
            xbs[b], out_hbm.at[ss, pl.ds(kk * NVG, NVG), wid], wsems[b])

    def transpose(b):
        def vbody(vg, carry):
            for j in range(8):
                col = jnp.full((16,), vg * 8 + j, jnp.int32)
                for g in range(8):
                    v = plsc.load_gather(segss[b], [row_idx[g], col])
                    xbs[b][vg, j, pl.ds(g * 16, 16)] = v
            return carry
        lax.fori_loop(0, NVG, vbody, 0)

    fill_seg_idx(0, 0)
    gather_cp(0).start()

    def step(jj, carry):
        for u in range(2):
            i = jj * 2 + u
            b = u
            other = 1 - u

            @pl.when(i + 1 < N_CHUNKS)
            def _():
                fill_seg_idx(i + 1, other)
                gather_cp(other).start()

            @pl.when(i >= 2)
            def _():
                write_cp(i - 2, b).wait()
            gather_cp(b).wait()
            transpose(b)
            write_cp(i, b).start()
        return carry

    lax.fori_loop(0, N_CHUNKS // 2, step, 0)
    write_cp(N_CHUNKS - 2, 0).wait()
    write_cp(N_CHUNKS - 1, 1).wait()


@jax.jit
def _embed(idx_t, tab_view):
    mesh = plsc.VectorSubcoreMesh(core_axis_name="c", subcore_axis_name="s")
    fn = pl.kernel(
        _emb_body,
        out_type=jax.ShapeDtypeStruct((SEQ, VOCAB // 8, NW, 8, BG),
                                      jnp.float32),
        mesh=mesh,
        scratch_types=[
            pltpu.VMEM((SEQ, BG), jnp.int32),
            pltpu.VMEM((BG,), jnp.int32),
            pltpu.VMEM((BG,), jnp.int32),
            pltpu.VMEM((BG, W), jnp.float32),
            pltpu.VMEM((BG, W), jnp.float32),
            pltpu.VMEM((NVG, 8, BG), jnp.float32),
            pltpu.VMEM((NVG, 8, BG), jnp.float32),
            pltpu.SemaphoreType.DMA,
            pltpu.SemaphoreType.DMA,
            pltpu.SemaphoreType.DMA,
            pltpu.SemaphoreType.DMA,
        ],
        compiler_params=pltpu.CompilerParams(use_tc_tiling_on_sc=False,
                                             needs_layout_passes=False),
    )
    return fn(tab_view, idx_t)


def kernel(inputs, targets, table):
    idx_t = inputs.astype(jnp.int32).T.reshape(SEQ, NW, BG)
    tab_view = table.reshape(NSEG, W)
    x = _embed(idx_t, tab_view)
    return x.transpose(2, 4, 0, 1, 3).reshape(BATCH, SEQ, VOCAB)


# speedup vs baseline: 4.3122x; 3.4031x over previous
import functools

import jax
import jax.numpy as jnp
from jax import lax
from jax.experimental import pallas as pl
from jax.experimental.pallas import tpu as pltpu
from jax.experimental.pallas import tpu_sc as plsc

VOCAB = 1000
BATCH = 4096
SEQ = 50

_info = plsc.get_sparse_core_info()
NC = _info.num_cores
NS = _info.num_subcores
NW = NC * NS
BG = BATCH // NW
W = 200
K = VOCAB // W
NVG = W // 8
N_CHUNKS = SEQ * K
NSEG = VOCAB * K


def _emb_body(tab_hbm, idx_hbm, out_hbm,
              idx_v, si0, si1, segs0, segs1, xb0, xb1,
              g0, g1, w0, w1):
    c = lax.axis_index("c")
    s = lax.axis_index("s")
    wid = s * NC + c

    pltpu.sync_copy(idx_hbm.at[:, wid], idx_v)

    sis = (si0, si1)
    segss = (segs0, segs1)
    xbs = (xb0, xb1)
    gsems = (g0, g1)
    wsems = (w0, w1)

    lane = lax.iota(jnp.int32, 16)
    row_idx = tuple(lane + (g * 16) for g in range(8))

    def fill_seg_idx(i, b):
        ss = i // K
        kk = i % K
        for g in range(8):
            r = idx_v[ss, pl.ds(g * 16, 16)]
            sis[b][pl.ds(g * 16, 16)] = r * K + kk

    def gather_cp(b):
        return pltpu.make_async_copy(tab_hbm.at[sis[b]], segss[b], gsems[b])

    def write_cp(i, b):
        ss = i // K
        kk = i % K
        return pltpu.make_async_copy(
            xbs[b], out_hbm.at[ss, pl.ds(kk * NVG, NVG), wid], wsems[b])

    def transpose(b):
        @functools.partial(plsc.parallel_loop, 0, NVG, unroll=2)
        def _(vg):
            for j in range(8):
                col = jnp.full((16,), vg * 8 + j, jnp.int32)
                for g in range(8):
                    v = plsc.load_gather(segss[b], [row_idx[g], col])
                    xbs[b][vg, j, pl.ds(g * 16, 16)] = v

    fill_seg_idx(0, 0)
    gather_cp(0).start()

    def step(jj, carry):
        for u in range(2):
            i = jj * 2 + u
            b = u
            other = 1 - u

            @pl.when(i + 1 < N_CHUNKS)
            def _():
                fill_seg_idx(i + 1, other)
                gather_cp(other).start()

            @pl.when(i >= 2)
            def _():
                write_cp(i - 2, b).wait()
            gather_cp(b).wait()
            transpose(b)
            write_cp(i, b).start()
        return carry

    lax.fori_loop(0, N_CHUNKS // 2, step, 0)
    write_cp(N_CHUNKS - 2, 0).wait()
    write_cp(N_CHUNKS - 1, 1).wait()


@jax.jit
def _embed(idx_t, tab_view):
    mesh = plsc.VectorSubcoreMesh(core_axis_name="c", subcore_axis_name="s")
    fn = pl.kernel(
        _emb_body,
        out_type=jax.ShapeDtypeStruct((SEQ, VOCAB // 8, NW, 8, BG),
                                      jnp.float32),
        mesh=mesh,
        scratch_types=[
            pltpu.VMEM((SEQ, BG), jnp.int32),
            pltpu.VMEM((BG,), jnp.int32),
            pltpu.VMEM((BG,), jnp.int32),
            pltpu.VMEM((BG, W), jnp.float32),
            pltpu.VMEM((BG, W), jnp.float32),
            pltpu.VMEM((NVG, 8, BG), jnp.float32),
            pltpu.VMEM((NVG, 8, BG), jnp.float32),
            pltpu.SemaphoreType.DMA,
            pltpu.SemaphoreType.DMA,
            pltpu.SemaphoreType.DMA,
            pltpu.SemaphoreType.DMA,
        ],
        compiler_params=pltpu.CompilerParams(use_tc_tiling_on_sc=False,
                                             needs_layout_passes=False),
    )
    return fn(tab_view, idx_t)


def kernel(inputs, targets, table):
    idx_t = inputs.astype(jnp.int32).T.reshape(SEQ, NW, BG)
    tab_view = table.reshape(NSEG, W)
    x = _embed(idx_t, tab_view)
    return x.transpose(2, 4, 0, 1, 3).reshape(BATCH, SEQ, VOCAB)
